# baseline (speedup 1.0000x reference)
import jax
import jax.numpy as jnp
from jax import lax
from jax.experimental import pallas as pl
from jax.experimental.pallas import tpu as pltpu
from jax.experimental.pallas import tpu_sc as plsc

N = 10000
E = 320000
D_IN = 128
D_HID = 128
D_OUT = 64

NC = 2
NS = 16
EPC = E // NC
EPW = EPC // NS
C = 80
NCH = EPW // C

_MESH = plsc.VectorSubcoreMesh(core_axis_name="c", subcore_axis_name="s")


def _gath_body(g_hbm, src_hbm, out_hbm, sidx, msg, sem):
    cid = lax.axis_index("c")
    sid = lax.axis_index("s")

    def chunk(i, _):
        base = pl.multiple_of(cid * EPC + sid * EPW + i * C, 8)
        pltpu.sync_copy(src_hbm.at[pl.ds(base, C)], sidx)
        pltpu.async_copy(g_hbm.at[sidx], msg, sem).wait()
        pltpu.sync_copy(msg, out_hbm.at[pl.ds(base, C)])
        return 0

    lax.fori_loop(0, NCH, chunk, 0)


_gath = pl.kernel(
    _gath_body,
    out_type=[jax.ShapeDtypeStruct((E, D_HID), jnp.float32)],
    mesh=_MESH,
    scratch_types=[
        pltpu.VMEM((C,), jnp.int32),
        pltpu.VMEM((C, D_HID), jnp.float32),
        pltpu.SemaphoreType.DMA,
    ],
)


EC = 2560
NEB = E // EC
BN = 1000


def _tc_agg_body(dst_ref, msg_ref, agg_ref, cnt_ref):
    @pl.when(pl.program_id(0) == 0)
    def _():
        agg_ref[...] = jnp.zeros_like(agg_ref)
        cnt_ref[...] = jnp.zeros_like(cnt_ref)

    one = jnp.ones((1, 16), jnp.float32)

    def edge(e, _):
        d = dst_ref[0, e]
        agg_ref[pl.ds(d, 1), :] += msg_ref[pl.ds(e, 1), :]
        cnt_ref[pl.ds(d, 1), :] += one
        return 0

    lax.fori_loop(0, EC, edge, 0)


_tc_agg = pl.pallas_call(
    _tc_agg_body,
    grid=(NEB,),
    in_specs=[
        pl.BlockSpec((1, EC), lambda i: (0, i), memory_space=pltpu.SMEM),
        pl.BlockSpec((EC, D_HID), lambda i: (i, 0)),
    ],
    out_specs=[
        pl.BlockSpec((N, D_HID), lambda i: (0, 0)),
        pl.BlockSpec((N, 16), lambda i: (0, 0)),
    ],
    out_shape=[
        jax.ShapeDtypeStruct((N, D_HID), jnp.float32),
        jax.ShapeDtypeStruct((N, 16), jnp.float32),
    ],
)


def _matT(a, w):
    return lax.dot_general(a, w, (((1,), (1,)), ((), ())),
                           preferred_element_type=jnp.float32)


def _inproj_body(x_ref, w_ref, b_ref, h0_ref, hr_ref):
    h0 = _matT(x_ref[...], w_ref[...]) + b_ref[...]
    h0_ref[...] = h0
    hr_ref[...] = jnp.maximum(h0, 0.0)


_inproj = pl.pallas_call(
    _inproj_body,
    grid=(N // BN,),
    in_specs=[
        pl.BlockSpec((BN, D_IN), lambda i: (i, 0)),
        pl.BlockSpec((D_HID, D_IN), lambda i: (0, 0)),
        pl.BlockSpec((1, D_HID), lambda i: (0, 0)),
    ],
    out_specs=[
        pl.BlockSpec((BN, D_HID), lambda i: (i, 0)),
        pl.BlockSpec((BN, D_HID), lambda i: (i, 0)),
    ],
    out_shape=[
        jax.ShapeDtypeStruct((N, D_HID), jnp.float32),
        jax.ShapeDtypeStruct((N, D_HID), jnp.float32),
    ],
)


def _sage_body(agg_ref, cnt_ref, h_ref, h0_ref, wl_ref, wr_ref, b_ref,
               g_ref, z_ref, hn_ref, gn_ref):
    inv = 1.0 / jnp.maximum(cnt_ref[:, 0:1], 1.0)
    mean = agg_ref[...] * inv
    z = _matT(mean, wl_ref[...]) + _matT(h_ref[...], wr_ref[...]) + b_ref[...]
    z_ref[...] = z
    hn = jnp.maximum(z, 0.0) + 0.2 * h0_ref[...]
    hn_ref[...] = hn
    gn_ref[...] = _matT(hn, g_ref[...])


_sage = pl.pallas_call(
    _sage_body,
    grid=(N // BN,),
    in_specs=[
        pl.BlockSpec((BN, D_HID), lambda i: (i, 0)),
        pl.BlockSpec((BN, 16), lambda i: (i, 0)),
        pl.BlockSpec((BN, D_HID), lambda i: (i, 0)),
        pl.BlockSpec((BN, D_HID), lambda i: (i, 0)),
        pl.BlockSpec((D_HID, D_HID), lambda i: (0, 0)),
        pl.BlockSpec((D_HID, D_HID), lambda i: (0, 0)),
        pl.BlockSpec((1, D_HID), lambda i: (0, 0)),
        pl.BlockSpec((D_HID, D_HID), lambda i: (0, 0)),
    ],
    out_specs=[
        pl.BlockSpec((BN, D_HID), lambda i: (i, 0)),
        pl.BlockSpec((BN, D_HID), lambda i: (i, 0)),
        pl.BlockSpec((BN, D_HID), lambda i: (i, 0)),
    ],
    out_shape=[
        jax.ShapeDtypeStruct((N, D_HID), jnp.float32),
        jax.ShapeDtypeStruct((N, D_HID), jnp.float32),
        jax.ShapeDtypeStruct((N, D_HID), jnp.float32),
    ],
)


def _final_body(z_ref, out_ref):
    z = z_ref[:, :D_OUT]
    z = z - jnp.max(z, axis=-1, keepdims=True)
    out_ref[...] = z - jnp.log(jnp.sum(jnp.exp(z), axis=-1, keepdims=True))


_final = pl.pallas_call(
    _final_body,
    grid=(N // BN,),
    in_specs=[pl.BlockSpec((BN, D_HID), lambda i: (i, 0))],
    out_specs=pl.BlockSpec((BN, D_OUT), lambda i: (i, 0)),
    out_shape=jax.ShapeDtypeStruct((N, D_OUT), jnp.float32),
)


def kernel(x, edge_index, W_in, b_in, Wl0, Wr0, b0, Wl1, Wr1, b1,
           Wl2, Wr2, b2):
    src = edge_index[0]
    dst2d = edge_index[1].reshape(1, E)

    h0, r = _inproj(x, W_in, b_in.reshape(1, -1))

    eye = jnp.eye(D_HID, dtype=jnp.float32)
    wl2p = jnp.zeros((D_HID, D_HID), jnp.float32).at[:D_OUT].set(Wl2)
    wr2p = jnp.zeros((D_HID, D_HID), jnp.float32).at[:D_OUT].set(Wr2)
    b2p = jnp.zeros((D_HID,), jnp.float32).at[:D_OUT].set(b2)

    def layer(g, h, wl, wr, b, gm):
        (msg,) = _gath(g, src)
        agg, cnt = _tc_agg(dst2d, msg)
        return _sage(agg, cnt, h, h0, wl, wr, b.reshape(1, -1), gm)

    _, h1, g1 = layer(r, r, Wl0, Wr0, b0, eye)
    _, h2, g2 = layer(g1, h1, Wl1, Wr1, b1, wl2p)
    z2, _, _ = layer(g2, h2, eye, wr2p, b2p, eye)
    return _final(z2)


# speedup vs baseline: 1.9262x; 1.9262x over previous
import jax
import jax.numpy as jnp
from jax import lax
from jax.experimental import pallas as pl
from jax.experimental.pallas import tpu as pltpu
from jax.experimental.pallas import tpu_sc as plsc

N = 10000
E = 320000
D_IN = 128
D_HID = 128
D_OUT = 64

NC = 2
NS = 16
EPC = E // NC
EPW = EPC // NS
C = 80
NCH = EPW // C

_MESH = plsc.VectorSubcoreMesh(core_axis_name="c", subcore_axis_name="s")


def _gath_body(g_hbm, src_hbm, out_hbm, sidx, msg, sem):
    cid = lax.axis_index("c")
    sid = lax.axis_index("s")

    def chunk(i, _):
        base = pl.multiple_of(cid * EPC + sid * EPW + i * C, 8)
        pltpu.sync_copy(src_hbm.at[pl.ds(base, C)], sidx)
        pltpu.async_copy(g_hbm.at[sidx], msg, sem).wait()
        pltpu.sync_copy(msg, out_hbm.at[pl.ds(base, C)])
        return 0

    lax.fori_loop(0, NCH, chunk, 0)


_gath = pl.kernel(
    _gath_body,
    out_type=[jax.ShapeDtypeStruct((E, D_HID), jnp.float32)],
    mesh=_MESH,
    scratch_types=[
        pltpu.VMEM((C,), jnp.int32),
        pltpu.VMEM((C, D_HID), jnp.float32),
        pltpu.SemaphoreType.DMA,
    ],
)


EC = 2560
NEB = E // EC
BN = 1000


def _tc_agg_body(dst_ref, msg_ref, agg_ref, cnt_ref):
    @pl.when(pl.program_id(0) == 0)
    def _():
        agg_ref[...] = jnp.zeros_like(agg_ref)
        cnt_ref[...] = jnp.zeros_like(cnt_ref)

    one = jnp.ones((1, 16), jnp.float32)

    def edge(e, _):
        d = dst_ref[0, e]
        agg_ref[pl.ds(d, 1), :] += msg_ref[pl.ds(e, 1), :]
        cnt_ref[pl.ds(d, 1), :] += one
        return 0

    lax.fori_loop(0, EC, edge, 0, unroll=8)


def _tc_agg_nc_body(dst_ref, msg_ref, agg_ref):
    @pl.when(pl.program_id(0) == 0)
    def _():
        agg_ref[...] = jnp.zeros_like(agg_ref)

    def edge(e, _):
        d = dst_ref[0, e]
        agg_ref[pl.ds(d, 1), :] += msg_ref[pl.ds(e, 1), :]
        return 0

    lax.fori_loop(0, EC, edge, 0, unroll=8)


_tc_agg_nc = pl.pallas_call(
    _tc_agg_nc_body,
    grid=(NEB,),
    in_specs=[
        pl.BlockSpec((1, EC), lambda i: (0, i), memory_space=pltpu.SMEM),
        pl.BlockSpec((EC, D_HID), lambda i: (i, 0)),
    ],
    out_specs=pl.BlockSpec((N, D_HID), lambda i: (0, 0)),
    out_shape=jax.ShapeDtypeStruct((N, D_HID), jnp.float32),
)


_tc_agg = pl.pallas_call(
    _tc_agg_body,
    grid=(NEB,),
    in_specs=[
        pl.BlockSpec((1, EC), lambda i: (0, i), memory_space=pltpu.SMEM),
        pl.BlockSpec((EC, D_HID), lambda i: (i, 0)),
    ],
    out_specs=[
        pl.BlockSpec((N, D_HID), lambda i: (0, 0)),
        pl.BlockSpec((N, 16), lambda i: (0, 0)),
    ],
    out_shape=[
        jax.ShapeDtypeStruct((N, D_HID), jnp.float32),
        jax.ShapeDtypeStruct((N, 16), jnp.float32),
    ],
)


def _matT(a, w):
    return lax.dot_general(a, w, (((1,), (1,)), ((), ())),
                           preferred_element_type=jnp.float32)


def _inproj_body(x_ref, w_ref, b_ref, h0_ref, hr_ref):
    h0 = _matT(x_ref[...], w_ref[...]) + b_ref[...]
    h0_ref[...] = h0
    hr_ref[...] = jnp.maximum(h0, 0.0)


_inproj = pl.pallas_call(
    _inproj_body,
    grid=(N // BN,),
    in_specs=[
        pl.BlockSpec((BN, D_IN), lambda i: (i, 0)),
        pl.BlockSpec((D_HID, D_IN), lambda i: (0, 0)),
        pl.BlockSpec((1, D_HID), lambda i: (0, 0)),
    ],
    out_specs=[
        pl.BlockSpec((BN, D_HID), lambda i: (i, 0)),
        pl.BlockSpec((BN, D_HID), lambda i: (i, 0)),
    ],
    out_shape=[
        jax.ShapeDtypeStruct((N, D_HID), jnp.float32),
        jax.ShapeDtypeStruct((N, D_HID), jnp.float32),
    ],
)


def _sage_body(agg_ref, cnt_ref, h_ref, h0_ref, wl_ref, wr_ref, b_ref,
               g_ref, z_ref, hn_ref, gn_ref):
    inv = 1.0 / jnp.maximum(cnt_ref[:, 0:1], 1.0)
    mean = agg_ref[...] * inv
    z = _matT(mean, wl_ref[...]) + _matT(h_ref[...], wr_ref[...]) + b_ref[...]
    z_ref[...] = z
    hn = jnp.maximum(z, 0.0) + 0.2 * h0_ref[...]
    hn_ref[...] = hn
    gn_ref[...] = _matT(hn, g_ref[...])


_sage = pl.pallas_call(
    _sage_body,
    grid=(N // BN,),
    in_specs=[
        pl.BlockSpec((BN, D_HID), lambda i: (i, 0)),
        pl.BlockSpec((BN, 16), lambda i: (i, 0)),
        pl.BlockSpec((BN, D_HID), lambda i: (i, 0)),
        pl.BlockSpec((BN, D_HID), lambda i: (i, 0)),
        pl.BlockSpec((D_HID, D_HID), lambda i: (0, 0)),
        pl.BlockSpec((D_HID, D_HID), lambda i: (0, 0)),
        pl.BlockSpec((1, D_HID), lambda i: (0, 0)),
        pl.BlockSpec((D_HID, D_HID), lambda i: (0, 0)),
    ],
    out_specs=[
        pl.BlockSpec((BN, D_HID), lambda i: (i, 0)),
        pl.BlockSpec((BN, D_HID), lambda i: (i, 0)),
        pl.BlockSpec((BN, D_HID), lambda i: (i, 0)),
    ],
    out_shape=[
        jax.ShapeDtypeStruct((N, D_HID), jnp.float32),
        jax.ShapeDtypeStruct((N, D_HID), jnp.float32),
        jax.ShapeDtypeStruct((N, D_HID), jnp.float32),
    ],
)


def _final_body(z_ref, out_ref):
    z = z_ref[:, :D_OUT]
    z = z - jnp.max(z, axis=-1, keepdims=True)
    out_ref[...] = z - jnp.log(jnp.sum(jnp.exp(z), axis=-1, keepdims=True))


_final = pl.pallas_call(
    _final_body,
    grid=(N // BN,),
    in_specs=[pl.BlockSpec((BN, D_HID), lambda i: (i, 0))],
    out_specs=pl.BlockSpec((BN, D_OUT), lambda i: (i, 0)),
    out_shape=jax.ShapeDtypeStruct((N, D_OUT), jnp.float32),
)


def kernel(x, edge_index, W_in, b_in, Wl0, Wr0, b0, Wl1, Wr1, b1,
           Wl2, Wr2, b2):
    src = edge_index[0]
    dst2d = edge_index[1].reshape(1, E)

    h0, r = _inproj(x, W_in, b_in.reshape(1, -1))

    eye = jnp.eye(D_HID, dtype=jnp.float32)
    wl2p = jnp.zeros((D_HID, D_HID), jnp.float32).at[:D_OUT].set(Wl2)
    wr2p = jnp.zeros((D_HID, D_HID), jnp.float32).at[:D_OUT].set(Wr2)
    b2p = jnp.zeros((D_HID,), jnp.float32).at[:D_OUT].set(b2)

    def layer(g, h, wl, wr, b, gm, cnt=None):
        (msg,) = _gath(g, src)
        if cnt is None:
            agg, cnt = _tc_agg(dst2d, msg)
        else:
            agg = _tc_agg_nc(dst2d, msg)
        return _sage(agg, cnt, h, h0, wl, wr, b.reshape(1, -1), gm), cnt

    (_, h1, g1), cnt = layer(r, r, Wl0, Wr0, b0, eye)
    (_, h2, g2), _ = layer(g1, h1, Wl1, Wr1, b1, wl2p, cnt)
    (z2, _, _), _ = layer(g2, h2, eye, wr2p, b2p, eye, cnt)
    return _final(z2)


# speedup vs baseline: 2.3423x; 1.2160x over previous
import jax
import jax.numpy as jnp
from jax import lax
from jax.experimental import pallas as pl
from jax.experimental.pallas import tpu as pltpu
from jax.experimental.pallas import tpu_sc as plsc

N = 10000
E = 320000
D_IN = 128
D_HID = 128
D_OUT = 64

NC = 2
NS = 16
EPC = E // NC
EPW = EPC // NS
C = 80
NCH = EPW // C

_MESH = plsc.VectorSubcoreMesh(core_axis_name="c", subcore_axis_name="s")


def _gath_body(g_hbm, src_hbm, out_hbm, sidx, msg, sem):
    cid = lax.axis_index("c")
    sid = lax.axis_index("s")

    def chunk(i, _):
        base = pl.multiple_of(cid * EPC + sid * EPW + i * C, 8)
        pltpu.sync_copy(src_hbm.at[pl.ds(base, C)], sidx)
        pltpu.async_copy(g_hbm.at[sidx], msg, sem).wait()
        pltpu.sync_copy(msg, out_hbm.at[pl.ds(base, C)])
        return 0

    lax.fori_loop(0, NCH, chunk, 0)


_gath = pl.kernel(
    _gath_body,
    out_type=[jax.ShapeDtypeStruct((E, D_HID), jnp.float32)],
    mesh=_MESH,
    scratch_types=[
        pltpu.VMEM((C,), jnp.int32),
        pltpu.VMEM((C, D_HID), jnp.float32),
        pltpu.SemaphoreType.DMA,
    ],
)


EC = 6400
NEB = E // EC
BN = 1000


def _tc_agg_body(dst_ref, msg_ref, agg_ref, cnt_ref, agg2_ref, cnt2_ref):
    @pl.when(pl.program_id(0) == 0)
    def _():
        agg_ref[...] = jnp.zeros_like(agg_ref)
        cnt_ref[...] = jnp.zeros_like(cnt_ref)
        agg2_ref[...] = jnp.zeros_like(agg2_ref)
        cnt2_ref[...] = jnp.zeros_like(cnt2_ref)

    one = jnp.ones((1, 16), jnp.float32)

    def pair(j, _):
        e = 2 * j
        d = dst_ref[0, e]
        d2 = dst_ref[0, e + 1]
        agg_ref[pl.ds(d, 1), :] += msg_ref[pl.ds(e, 1), :]
        agg2_ref[pl.ds(d2, 1), :] += msg_ref[pl.ds(e + 1, 1), :]
        cnt_ref[pl.ds(d, 1), :] += one
        cnt2_ref[pl.ds(d2, 1), :] += one
        return 0

    lax.fori_loop(0, EC // 2, pair, 0, unroll=4)

    @pl.when(pl.program_id(0) == NEB - 1)
    def _():
        agg_ref[...] += agg2_ref[...]
        cnt_ref[...] += cnt2_ref[...]


def _tc_agg_nc_body(dst_ref, msg_ref, agg_ref, agg2_ref):
    @pl.when(pl.program_id(0) == 0)
    def _():
        agg_ref[...] = jnp.zeros_like(agg_ref)
        agg2_ref[...] = jnp.zeros_like(agg2_ref)

    def pair(j, _):
        e = 2 * j
        d = dst_ref[0, e]
        d2 = dst_ref[0, e + 1]
        agg_ref[pl.ds(d, 1), :] += msg_ref[pl.ds(e, 1), :]
        agg2_ref[pl.ds(d2, 1), :] += msg_ref[pl.ds(e + 1, 1), :]
        return 0

    lax.fori_loop(0, EC // 2, pair, 0, unroll=4)

    @pl.when(pl.program_id(0) == NEB - 1)
    def _():
        agg_ref[...] += agg2_ref[...]


_tc_agg_nc = pl.pallas_call(
    _tc_agg_nc_body,
    grid=(NEB,),
    in_specs=[
        pl.BlockSpec((1, EC), lambda i: (0, i), memory_space=pltpu.SMEM),
        pl.BlockSpec((EC, D_HID), lambda i: (i, 0)),
    ],
    out_specs=[
        pl.BlockSpec((N, D_HID), lambda i: (0, 0)),
        pl.BlockSpec((N, D_HID), lambda i: (0, 0)),
    ],
    out_shape=[
        jax.ShapeDtypeStruct((N, D_HID), jnp.float32),
        jax.ShapeDtypeStruct((N, D_HID), jnp.float32),
    ],
)


_tc_agg = pl.pallas_call(
    _tc_agg_body,
    grid=(NEB,),
    in_specs=[
        pl.BlockSpec((1, EC), lambda i: (0, i), memory_space=pltpu.SMEM),
        pl.BlockSpec((EC, D_HID), lambda i: (i, 0)),
    ],
    out_specs=[
        pl.BlockSpec((N, D_HID), lambda i: (0, 0)),
        pl.BlockSpec((N, 16), lambda i: (0, 0)),
        pl.BlockSpec((N, D_HID), lambda i: (0, 0)),
        pl.BlockSpec((N, 16), lambda i: (0, 0)),
    ],
    out_shape=[
        jax.ShapeDtypeStruct((N, D_HID), jnp.float32),
        jax.ShapeDtypeStruct((N, 16), jnp.float32),
        jax.ShapeDtypeStruct((N, D_HID), jnp.float32),
        jax.ShapeDtypeStruct((N, 16), jnp.float32),
    ],
)


def _matT(a, w):
    return lax.dot_general(a, w, (((1,), (1,)), ((), ())),
                           preferred_element_type=jnp.float32)


def _inproj_body(x_ref, w_ref, b_ref, h0_ref, hr_ref):
    h0 = _matT(x_ref[...], w_ref[...]) + b_ref[...]
    h0_ref[...] = h0
    hr_ref[...] = jnp.maximum(h0, 0.0)


_inproj = pl.pallas_call(
    _inproj_body,
    grid=(N // BN,),
    in_specs=[
        pl.BlockSpec((BN, D_IN), lambda i: (i, 0)),
        pl.BlockSpec((D_HID, D_IN), lambda i: (0, 0)),
        pl.BlockSpec((1, D_HID), lambda i: (0, 0)),
    ],
    out_specs=[
        pl.BlockSpec((BN, D_HID), lambda i: (i, 0)),
        pl.BlockSpec((BN, D_HID), lambda i: (i, 0)),
    ],
    out_shape=[
        jax.ShapeDtypeStruct((N, D_HID), jnp.float32),
        jax.ShapeDtypeStruct((N, D_HID), jnp.float32),
    ],
)


def _sage_body(agg_ref, cnt_ref, h_ref, h0_ref, wl_ref, wr_ref, b_ref,
               g_ref, z_ref, hn_ref, gn_ref):
    inv = 1.0 / jnp.maximum(cnt_ref[:, 0:1], 1.0)
    mean = agg_ref[...] * inv
    z = _matT(mean, wl_ref[...]) + _matT(h_ref[...], wr_ref[...]) + b_ref[...]
    z_ref[...] = z
    hn = jnp.maximum(z, 0.0) + 0.2 * h0_ref[...]
    hn_ref[...] = hn
    gn_ref[...] = _matT(hn, g_ref[...])


_sage = pl.pallas_call(
    _sage_body,
    grid=(N // BN,),
    in_specs=[
        pl.BlockSpec((BN, D_HID), lambda i: (i, 0)),
        pl.BlockSpec((BN, 16), lambda i: (i, 0)),
        pl.BlockSpec((BN, D_HID), lambda i: (i, 0)),
        pl.BlockSpec((BN, D_HID), lambda i: (i, 0)),
        pl.BlockSpec((D_HID, D_HID), lambda i: (0, 0)),
        pl.BlockSpec((D_HID, D_HID), lambda i: (0, 0)),
        pl.BlockSpec((1, D_HID), lambda i: (0, 0)),
        pl.BlockSpec((D_HID, D_HID), lambda i: (0, 0)),
    ],
    out_specs=[
        pl.BlockSpec((BN, D_HID), lambda i: (i, 0)),
        pl.BlockSpec((BN, D_HID), lambda i: (i, 0)),
        pl.BlockSpec((BN, D_HID), lambda i: (i, 0)),
    ],
    out_shape=[
        jax.ShapeDtypeStruct((N, D_HID), jnp.float32),
        jax.ShapeDtypeStruct((N, D_HID), jnp.float32),
        jax.ShapeDtypeStruct((N, D_HID), jnp.float32),
    ],
)


def _final_body(z_ref, out_ref):
    z = z_ref[:, :D_OUT]
    z = z - jnp.max(z, axis=-1, keepdims=True)
    out_ref[...] = z - jnp.log(jnp.sum(jnp.exp(z), axis=-1, keepdims=True))


_final = pl.pallas_call(
    _final_body,
    grid=(N // BN,),
    in_specs=[pl.BlockSpec((BN, D_HID), lambda i: (i, 0))],
    out_specs=pl.BlockSpec((BN, D_OUT), lambda i: (i, 0)),
    out_shape=jax.ShapeDtypeStruct((N, D_OUT), jnp.float32),
)


def kernel(x, edge_index, W_in, b_in, Wl0, Wr0, b0, Wl1, Wr1, b1,
           Wl2, Wr2, b2):
    src = edge_index[0]
    dst2d = edge_index[1].reshape(1, E)

    h0, r = _inproj(x, W_in, b_in.reshape(1, -1))

    eye = jnp.eye(D_HID, dtype=jnp.float32)
    wl2p = jnp.zeros((D_HID, D_HID), jnp.float32).at[:D_OUT].set(Wl2)
    wr2p = jnp.zeros((D_HID, D_HID), jnp.float32).at[:D_OUT].set(Wr2)
    b2p = jnp.zeros((D_HID,), jnp.float32).at[:D_OUT].set(b2)

    def layer(g, h, wl, wr, b, gm, cnt=None):
        (msg,) = _gath(g, src)
        if cnt is None:
            agg, cnt, _, _ = _tc_agg(dst2d, msg)
        else:
            agg, _ = _tc_agg_nc(dst2d, msg)
        return _sage(agg, cnt, h, h0, wl, wr, b.reshape(1, -1), gm), cnt

    (_, h1, g1), cnt = layer(r, r, Wl0, Wr0, b0, eye)
    (_, h2, g2), _ = layer(g1, h1, Wl1, Wr1, b1, wl2p, cnt)
    (z2, _, _), _ = layer(g2, h2, eye, wr2p, b2p, eye, cnt)
    return _final(z2)


# speedup vs baseline: 2.6236x; 1.1201x over previous
import jax
import jax.numpy as jnp
from jax import lax
from jax.experimental import pallas as pl
from jax.experimental.pallas import tpu as pltpu
from jax.experimental.pallas import tpu_sc as plsc

N = 10000
E = 320000
D_IN = 128
D_HID = 128
D_OUT = 64

NC = 2
NS = 16
EPC = E // NC
EPW = EPC // NS
C = 80
NCH = EPW // C

_MESH = plsc.VectorSubcoreMesh(core_axis_name="c", subcore_axis_name="s")


def _gath_body(g_hbm, src_hbm, out_hbm, sidx, msg, sem):
    cid = lax.axis_index("c")
    sid = lax.axis_index("s")

    def chunk(i, _):
        base = pl.multiple_of(cid * EPC + sid * EPW + i * C, 8)
        pltpu.sync_copy(src_hbm.at[pl.ds(base, C)], sidx)
        pltpu.async_copy(g_hbm.at[sidx], msg, sem).wait()
        pltpu.sync_copy(msg, out_hbm.at[pl.ds(base, C)])
        return 0

    lax.fori_loop(0, NCH, chunk, 0)


_gath = pl.kernel(
    _gath_body,
    out_type=[jax.ShapeDtypeStruct((E, D_HID), jnp.float32)],
    mesh=_MESH,
    scratch_types=[
        pltpu.VMEM((C,), jnp.int32),
        pltpu.VMEM((C, D_HID), jnp.float32),
        pltpu.SemaphoreType.DMA,
    ],
)


EC = 6400
NEB = E // EC
BN = 1000


def _tc_agg_body(dst_ref, msg_ref, agg_ref, cnt_ref, agg2_ref, cnt2_ref,
                 agg3_ref, agg4_ref):
    @pl.when(pl.program_id(0) == 0)
    def _():
        agg_ref[...] = jnp.zeros_like(agg_ref)
        cnt_ref[...] = jnp.zeros_like(cnt_ref)
        agg2_ref[...] = jnp.zeros_like(agg2_ref)
        cnt2_ref[...] = jnp.zeros_like(cnt2_ref)
        agg3_ref[...] = jnp.zeros_like(agg3_ref)
        agg4_ref[...] = jnp.zeros_like(agg4_ref)

    one = jnp.ones((1, 16), jnp.float32)

    def quad(j, _):
        e = 4 * j
        d0 = dst_ref[0, e]
        d1 = dst_ref[0, e + 1]
        d2 = dst_ref[0, e + 2]
        d3 = dst_ref[0, e + 3]
        agg_ref[pl.ds(d0, 1), :] += msg_ref[pl.ds(e, 1), :]
        agg2_ref[pl.ds(d1, 1), :] += msg_ref[pl.ds(e + 1, 1), :]
        agg3_ref[pl.ds(d2, 1), :] += msg_ref[pl.ds(e + 2, 1), :]
        agg4_ref[pl.ds(d3, 1), :] += msg_ref[pl.ds(e + 3, 1), :]
        cnt_ref[pl.ds(d0, 1), :] += one
        cnt2_ref[pl.ds(d1, 1), :] += one
        cnt_ref[pl.ds(d2, 1), :] += one
        cnt2_ref[pl.ds(d3, 1), :] += one
        return 0

    lax.fori_loop(0, EC // 4, quad, 0, unroll=2)

    @pl.when(pl.program_id(0) == NEB - 1)
    def _():
        agg_ref[...] += agg2_ref[...] + agg3_ref[...] + agg4_ref[...]
        cnt_ref[...] += cnt2_ref[...]


def _tc_agg_nc_body(dst_ref, msg_ref, agg_ref, agg2_ref, agg3_ref,
                    agg4_ref):
    @pl.when(pl.program_id(0) == 0)
    def _():
        agg_ref[...] = jnp.zeros_like(agg_ref)
        agg2_ref[...] = jnp.zeros_like(agg2_ref)
        agg3_ref[...] = jnp.zeros_like(agg3_ref)
        agg4_ref[...] = jnp.zeros_like(agg4_ref)

    def quad(j, _):
        e = 4 * j
        d0 = dst_ref[0, e]
        d1 = dst_ref[0, e + 1]
        d2 = dst_ref[0, e + 2]
        d3 = dst_ref[0, e + 3]
        agg_ref[pl.ds(d0, 1), :] += msg_ref[pl.ds(e, 1), :]
        agg2_ref[pl.ds(d1, 1), :] += msg_ref[pl.ds(e + 1, 1), :]
        agg3_ref[pl.ds(d2, 1), :] += msg_ref[pl.ds(e + 2, 1), :]
        agg4_ref[pl.ds(d3, 1), :] += msg_ref[pl.ds(e + 3, 1), :]
        return 0

    lax.fori_loop(0, EC // 4, quad, 0, unroll=2)

    @pl.when(pl.program_id(0) == NEB - 1)
    def _():
        agg_ref[...] += agg2_ref[...] + agg3_ref[...] + agg4_ref[...]


_tc_agg_nc = pl.pallas_call(
    _tc_agg_nc_body,
    grid=(NEB,),
    in_specs=[
        pl.BlockSpec((1, EC), lambda i: (0, i), memory_space=pltpu.SMEM),
        pl.BlockSpec((EC, D_HID), lambda i: (i, 0)),
    ],
    out_specs=[
        pl.BlockSpec((N, D_HID), lambda i: (0, 0)),
        pl.BlockSpec((N, D_HID), lambda i: (0, 0)),
        pl.BlockSpec((N, D_HID), lambda i: (0, 0)),
        pl.BlockSpec((N, D_HID), lambda i: (0, 0)),
    ],
    out_shape=[
        jax.ShapeDtypeStruct((N, D_HID), jnp.float32),
        jax.ShapeDtypeStruct((N, D_HID), jnp.float32),
        jax.ShapeDtypeStruct((N, D_HID), jnp.float32),
        jax.ShapeDtypeStruct((N, D_HID), jnp.float32),
    ],
)


_tc_agg = pl.pallas_call(
    _tc_agg_body,
    grid=(NEB,),
    in_specs=[
        pl.BlockSpec((1, EC), lambda i: (0, i), memory_space=pltpu.SMEM),
        pl.BlockSpec((EC, D_HID), lambda i: (i, 0)),
    ],
    out_specs=[
        pl.BlockSpec((N, D_HID), lambda i: (0, 0)),
        pl.BlockSpec((N, 16), lambda i: (0, 0)),
        pl.BlockSpec((N, D_HID), lambda i: (0, 0)),
        pl.BlockSpec((N, 16), lambda i: (0, 0)),
        pl.BlockSpec((N, D_HID), lambda i: (0, 0)),
        pl.BlockSpec((N, D_HID), lambda i: (0, 0)),
    ],
    out_shape=[
        jax.ShapeDtypeStruct((N, D_HID), jnp.float32),
        jax.ShapeDtypeStruct((N, 16), jnp.float32),
        jax.ShapeDtypeStruct((N, D_HID), jnp.float32),
        jax.ShapeDtypeStruct((N, 16), jnp.float32),
        jax.ShapeDtypeStruct((N, D_HID), jnp.float32),
        jax.ShapeDtypeStruct((N, D_HID), jnp.float32),
    ],
)


def _matT(a, w):
    return lax.dot_general(a, w, (((1,), (1,)), ((), ())),
                           preferred_element_type=jnp.float32)


def _inproj_body(x_ref, w_ref, b_ref, h0_ref, hr_ref):
    h0 = _matT(x_ref[...], w_ref[...]) + b_ref[...]
    h0_ref[...] = h0
    hr_ref[...] = jnp.maximum(h0, 0.0)


_inproj = pl.pallas_call(
    _inproj_body,
    grid=(N // BN,),
    in_specs=[
        pl.BlockSpec((BN, D_IN), lambda i: (i, 0)),
        pl.BlockSpec((D_HID, D_IN), lambda i: (0, 0)),
        pl.BlockSpec((1, D_HID), lambda i: (0, 0)),
    ],
    out_specs=[
        pl.BlockSpec((BN, D_HID), lambda i: (i, 0)),
        pl.BlockSpec((BN, D_HID), lambda i: (i, 0)),
    ],
    out_shape=[
        jax.ShapeDtypeStruct((N, D_HID), jnp.float32),
        jax.ShapeDtypeStruct((N, D_HID), jnp.float32),
    ],
)


def _sage_body(agg_ref, cnt_ref, h_ref, h0_ref, wl_ref, wr_ref, b_ref,
               g_ref, z_ref, hn_ref, gn_ref):
    inv = 1.0 / jnp.maximum(cnt_ref[:, 0:1], 1.0)
    mean = agg_ref[...] * inv
    z = _matT(mean, wl_ref[...]) + _matT(h_ref[...], wr_ref[...]) + b_ref[...]
    z_ref[...] = z
    hn = jnp.maximum(z, 0.0) + 0.2 * h0_ref[...]
    hn_ref[...] = hn
    gn_ref[...] = _matT(hn, g_ref[...])


_sage = pl.pallas_call(
    _sage_body,
    grid=(N // BN,),
    in_specs=[
        pl.BlockSpec((BN, D_HID), lambda i: (i, 0)),
        pl.BlockSpec((BN, 16), lambda i: (i, 0)),
        pl.BlockSpec((BN, D_HID), lambda i: (i, 0)),
        pl.BlockSpec((BN, D_HID), lambda i: (i, 0)),
        pl.BlockSpec((D_HID, D_HID), lambda i: (0, 0)),
        pl.BlockSpec((D_HID, D_HID), lambda i: (0, 0)),
        pl.BlockSpec((1, D_HID), lambda i: (0, 0)),
        pl.BlockSpec((D_HID, D_HID), lambda i: (0, 0)),
    ],
    out_specs=[
        pl.BlockSpec((BN, D_HID), lambda i: (i, 0)),
        pl.BlockSpec((BN, D_HID), lambda i: (i, 0)),
        pl.BlockSpec((BN, D_HID), lambda i: (i, 0)),
    ],
    out_shape=[
        jax.ShapeDtypeStruct((N, D_HID), jnp.float32),
        jax.ShapeDtypeStruct((N, D_HID), jnp.float32),
        jax.ShapeDtypeStruct((N, D_HID), jnp.float32),
    ],
)


def _final_body(z_ref, out_ref):
    z = z_ref[:, :D_OUT]
    z = z - jnp.max(z, axis=-1, keepdims=True)
    out_ref[...] = z - jnp.log(jnp.sum(jnp.exp(z), axis=-1, keepdims=True))


_final = pl.pallas_call(
    _final_body,
    grid=(N // BN,),
    in_specs=[pl.BlockSpec((BN, D_HID), lambda i: (i, 0))],
    out_specs=pl.BlockSpec((BN, D_OUT), lambda i: (i, 0)),
    out_shape=jax.ShapeDtypeStruct((N, D_OUT), jnp.float32),
)


def kernel(x, edge_index, W_in, b_in, Wl0, Wr0, b0, Wl1, Wr1, b1,
           Wl2, Wr2, b2):
    src = edge_index[0]
    dst2d = edge_index[1].reshape(1, E)

    h0, r = _inproj(x, W_in, b_in.reshape(1, -1))

    eye = jnp.eye(D_HID, dtype=jnp.float32)
    wl2p = jnp.zeros((D_HID, D_HID), jnp.float32).at[:D_OUT].set(Wl2)
    wr2p = jnp.zeros((D_HID, D_HID), jnp.float32).at[:D_OUT].set(Wr2)
    b2p = jnp.zeros((D_HID,), jnp.float32).at[:D_OUT].set(b2)

    def layer(g, h, wl, wr, b, gm, cnt=None):
        (msg,) = _gath(g, src)
        if cnt is None:
            agg, cnt, _, _, _, _ = _tc_agg(dst2d, msg)
        else:
            agg, _, _, _ = _tc_agg_nc(dst2d, msg)
        return _sage(agg, cnt, h, h0, wl, wr, b.reshape(1, -1), gm), cnt

    (_, h1, g1), cnt = layer(r, r, Wl0, Wr0, b0, eye)
    (_, h2, g2), _ = layer(g1, h1, Wl1, Wr1, b1, wl2p, cnt)
    (z2, _, _), _ = layer(g2, h2, eye, wr2p, b2p, eye, cnt)
    return _final(z2)
